# baseline (speedup 1.0000x reference)
import functools

import jax
import jax.numpy as jnp
from jax import lax
from jax.experimental import pallas as pl
from jax.experimental.pallas import tpu as pltpu
from jax.experimental.pallas import tpu_sc as plsc

_N, _E, _DF, _DE, _H, _OUT, _G = 10000, 320000, 128, 16, 128, 64, 64
_NC, _NS = 2, 16
_NW = _NC * _NS
_CH = 128
_NCHUNK = _E // _CH
_MAXJ = -(-_NCHUNK // _NW)
_NPAD = 10240
_RPT = _NPAD // _NS

_K = 2


def _make_sc_body(linear):
    IC = 4

    def body(g_hbm, src_hbm, dst_hbm, z_hbm, out_hbm, sidx, didx, rows,
             accum, isem, gsem, ssem):
        c = lax.axis_index("c")
        s = lax.axis_index("s")
        w = s * _NC + c
        row0 = s * _RPT
        pltpu.sync_copy(z_hbm, accum.at[pl.ds(row0, _RPT), :])
        plsc.subcore_barrier()

        n_groups = -(-_MAXJ // _K)

        @pl.loop(0, n_groups)
        def group(gi):
            cids = []
            for b in range(_K):
                cid = (gi * _K + b) * _NW + w
                cids.append(cid)

                @pl.when(cid < _NCHUNK)
                def _(b=b, cid=cid):
                    base = cid * _CH
                    if not linear:
                        pltpu.async_copy(src_hbm.at[pl.ds(base, _CH)],
                                         sidx.at[b], isem.at[b])
                    pltpu.async_copy(dst_hbm.at[pl.ds(base, _CH)],
                                     didx.at[b], isem.at[b])

            for b in range(_K):
                cid = cids[b]

                @pl.when(cid < _NCHUNK)
                def _(b=b, cid=cid):
                    if not linear:
                        pltpu.make_async_copy(src_hbm.at[pl.ds(0, _CH)],
                                              sidx.at[b], isem.at[b]).wait()
                    pltpu.make_async_copy(dst_hbm.at[pl.ds(0, _CH)],
                                          didx.at[b], isem.at[b]).wait()
                    if linear:
                        base = cid * _CH
                        pltpu.async_copy(g_hbm.at[pl.ds(base, _CH), :],
                                         rows.at[b], gsem.at[b])
                    else:
                        pltpu.async_copy(g_hbm.at[sidx.at[b]], rows.at[b],
                                         gsem.at[b])

                @pl.when(cid < _NCHUNK)
                def _(b=b):
                    if linear:
                        pltpu.make_async_copy(g_hbm.at[pl.ds(0, _CH), :],
                                              rows.at[b], gsem.at[b]).wait()
                    else:
                        pltpu.make_async_copy(g_hbm.at[sidx.at[b]],
                                              rows.at[b], gsem.at[b]).wait()
                    pltpu.async_copy(rows.at[b], accum.at[didx.at[b]],
                                     ssem.at[b], add=True)

            for b in range(_K):
                cid = cids[b]

                @pl.when(cid < _NCHUNK)
                def _(b=b):
                    pltpu.make_async_copy(rows.at[b], accum.at[didx.at[b]],
                                          ssem.at[b]).wait()
        plsc.subcore_barrier()
        pltpu.sync_copy(accum.at[pl.ds(row0, _RPT), :],
                        out_hbm.at[c, pl.ds(row0, _RPT), :])

    return body


@functools.cache
def _sc_aggr_kernel(linear=False):
    return pl.kernel(
        _make_sc_body(linear),
        out_type=jax.ShapeDtypeStruct((_NC, _NPAD, _DF), jnp.float32),
        mesh=plsc.VectorSubcoreMesh(core_axis_name="c", subcore_axis_name="s"),
        scratch_types=[
            pltpu.VMEM((_K, _CH), jnp.int32),
            pltpu.VMEM((_K, _CH), jnp.int32),
            pltpu.VMEM((_K, _CH, _DF), jnp.float32),
            pltpu.VMEM_SHARED((_NPAD, _DF), jnp.float32),
            pltpu.SemaphoreType.DMA((4,)),
            pltpu.SemaphoreType.DMA((_K,)),
            pltpu.SemaphoreType.DMA((_K,)),
        ],
    )


_BN = 1000
_EB = 4000


def _relu_body(x_ref, gx_ref):
    gx_ref[...] = jnp.maximum(x_ref[...], 0.0)


def _relu_x(x):
    return pl.pallas_call(
        _relu_body,
        grid=(_N // _BN,),
        in_specs=[pl.BlockSpec((_BN, _DF), lambda i: (i, 0))],
        out_specs=pl.BlockSpec((_BN, _DF), lambda i: (i, 0)),
        out_shape=jax.ShapeDtypeStruct((_N, _DF), jnp.float32),
    )(x)


def _eapad_body(ea_ref, eap_ref):
    eap_ref[...] = jnp.concatenate(
        [jnp.maximum(ea_ref[...], 0.0),
         jnp.zeros((_EB, _DF - _DE), jnp.float32)], axis=1)


def _eapad(ea):
    return pl.pallas_call(
        _eapad_body,
        grid=(_E // _EB,),
        in_specs=[pl.BlockSpec((_EB, _DE), lambda i: (i, 0))],
        out_specs=pl.BlockSpec((_EB, _DF), lambda i: (i, 0)),
        out_shape=jax.ShapeDtypeStruct((_E, _DF), jnp.float32),
    )(ea)


def _layer_body(h_ref, p_ref, es_ref, W1a_ref, W1b_ref, b1_ref, W2_ref,
                b2_ref, o_ref):
    z = h_ref[...] + p_ref[0] + p_ref[1]
    es = es_ref[0] + es_ref[1]
    t = jnp.dot(z, W1a_ref[...], preferred_element_type=jnp.float32)
    t = t + jnp.dot(es, W1b_ref[...], preferred_element_type=jnp.float32)
    t = jnp.maximum(t + b1_ref[...], 0.0)
    o = jnp.dot(t, W2_ref[...], preferred_element_type=jnp.float32)
    o_ref[...] = jnp.maximum(o + b2_ref[...], 0.0)


def _layer(h, p, es, W1a, W1b, b1, W2, b2):
    return pl.pallas_call(
        _layer_body,
        grid=(_N // _BN,),
        in_specs=[
            pl.BlockSpec((_BN, _DF), lambda i: (i, 0)),
            pl.BlockSpec((_NC, _BN, _DF), lambda i: (0, i, 0)),
            pl.BlockSpec((_NC, _BN, _DF), lambda i: (0, i, 0)),
            pl.BlockSpec((_H, _H), lambda i: (0, 0)),
            pl.BlockSpec((_DF, _H), lambda i: (0, 0)),
            pl.BlockSpec((1, _H), lambda i: (0, 0)),
            pl.BlockSpec((_H, _H), lambda i: (0, 0)),
            pl.BlockSpec((1, _H), lambda i: (0, 0)),
        ],
        out_specs=pl.BlockSpec((_BN, _H), lambda i: (i, 0)),
        out_shape=jax.ShapeDtypeStruct((_N, _H), jnp.float32),
    )(h, p, es, W1a, W1b, b1, W2, b2)


def _pool_body(h_ref, b_ref, Wl_ref, bl_ref, o_ref, acc, cnt):
    i = pl.program_id(0)

    @pl.when(i == 0)
    def _():
        acc[...] = jnp.zeros_like(acc)
        cnt[...] = jnp.zeros_like(cnt)

    bvec = b_ref[0, 0]
    onehot = (bvec[:, None]
              == lax.broadcasted_iota(jnp.int32, (1, _G), 1)).astype(jnp.float32)
    acc[...] += lax.dot_general(onehot, h_ref[...], (((0,), (0,)), ((), ())),
                                preferred_element_type=jnp.float32)
    ones = jnp.ones((_BN, _H), jnp.float32)
    cnt[...] += lax.dot_general(onehot, ones, (((0,), (0,)), ((), ())),
                                preferred_element_type=jnp.float32)

    @pl.when(i == _N // _BN - 1)
    def _():
        pooled = acc[...] / jnp.maximum(cnt[...], 1.0)
        o_ref[...] = (jnp.dot(pooled, Wl_ref[...],
                              preferred_element_type=jnp.float32)
                      + bl_ref[...])


def _pool(h, batch3d, W_lin, b_lin2):
    return pl.pallas_call(
        _pool_body,
        grid=(_N // _BN,),
        in_specs=[
            pl.BlockSpec((_BN, _H), lambda i: (i, 0)),
            pl.BlockSpec((1, 1, _BN), lambda i: (i, 0, 0)),
            pl.BlockSpec((_H, _OUT), lambda i: (0, 0)),
            pl.BlockSpec((1, _OUT), lambda i: (0, 0)),
        ],
        out_specs=pl.BlockSpec((_G, _OUT), lambda i: (0, 0)),
        out_shape=jax.ShapeDtypeStruct((_G, _OUT), jnp.float32),
        scratch_shapes=[
            pltpu.VMEM((_G, _H), jnp.float32),
            pltpu.VMEM((_G, _H), jnp.float32),
        ],
    )(h, batch3d, W_lin, b_lin2)


def kernel(x, edge_index, edge_attr, batch,
           W1_0, b1_0, W2_0, b2_0,
           W1_1, b1_1, W2_1, b2_1,
           W1_2, b1_2, W2_2, b2_2,
           W_lin, b_lin):
    g0 = _relu_x(x)
    ea_pad = _eapad(edge_attr)

    z128 = jnp.zeros((_RPT, _DF), jnp.float32)
    src = edge_index[0]
    dst = edge_index[1]

    es = _sc_aggr_kernel(True)(ea_pad, dst, dst, z128)

    h = x
    g = g0
    for (W1, b1, W2, b2) in ((W1_0, b1_0, W2_0, b2_0),
                             (W1_1, b1_1, W2_1, b2_1),
                             (W1_2, b1_2, W2_2, b2_2)):
        p = _sc_aggr_kernel()(g, src, dst, z128)
        W1b_pad = jnp.concatenate(
            [W1[_H:], jnp.zeros((_DF - _DE, _H), jnp.float32)], axis=0)
        h = _layer(h, p, es, W1[:_H], W1b_pad, b1.reshape(1, _H),
                   W2, b2.reshape(1, _H))
        g = h

    batch3d = batch.reshape(_N // _BN, 1, _BN)
    return _pool(h, batch3d, W_lin, b_lin.reshape(1, _OUT))


# speedup vs baseline: 1.0020x; 1.0020x over previous
import functools

import jax
import jax.numpy as jnp
from jax import lax
from jax.experimental import pallas as pl
from jax.experimental.pallas import tpu as pltpu
from jax.experimental.pallas import tpu_sc as plsc

_N, _E, _DF, _DE, _H, _OUT, _G = 10000, 320000, 128, 16, 128, 64, 64
_NC, _NS = 2, 16
_NW = _NC * _NS
_CH = 128
_NCHUNK = _E // _CH
_MAXJ = -(-_NCHUNK // _NW)
_NPAD = 10240
_RPT = _NPAD // _NS

_K = 2


def _make_sc_body(linear):
    IC = 4

    def body(g_hbm, src_hbm, dst_hbm, z_hbm, out_hbm, sidx, didx, rows,
             accum, isem, gsem, ssem):
        c = lax.axis_index("c")
        s = lax.axis_index("s")
        w = s * _NC + c
        row0 = s * _RPT
        pltpu.sync_copy(z_hbm, accum.at[pl.ds(row0, _RPT), :])
        plsc.subcore_barrier()

        def fire_idx(j):
            slot = j % 2
            cid = j * _NW + w

            @pl.when(cid < _NCHUNK)
            def _():
                base = cid * _CH
                if not linear:
                    pltpu.async_copy(src_hbm.at[pl.ds(base, _CH)],
                                     sidx.at[slot], isem.at[slot])
                pltpu.async_copy(dst_hbm.at[pl.ds(base, _CH)],
                                 didx.at[slot], isem.at[slot])

        fire_idx(0)
        for j in range(_MAXJ + 1):
            b = j % 2
            cid = j * _NW + w

            if j < _MAXJ:
                @pl.when(cid < _NCHUNK)
                def _(b=b, cid=cid):
                    if not linear:
                        pltpu.make_async_copy(src_hbm.at[pl.ds(0, _CH)],
                                              sidx.at[b], isem.at[b]).wait()
                    pltpu.make_async_copy(dst_hbm.at[pl.ds(0, _CH)],
                                          didx.at[b], isem.at[b]).wait()
                    if linear:
                        base = cid * _CH
                        pltpu.async_copy(g_hbm.at[pl.ds(base, _CH), :],
                                         rows.at[b], gsem.at[b])
                        pltpu.make_async_copy(g_hbm.at[pl.ds(0, _CH), :],
                                              rows.at[b], gsem.at[b]).wait()
                    else:
                        pltpu.async_copy(g_hbm.at[sidx.at[b]], rows.at[b],
                                         gsem.at[b])
                        pltpu.make_async_copy(g_hbm.at[sidx.at[b]],
                                              rows.at[b], gsem.at[b]).wait()

            if j >= 1:
                @pl.when(((j - 1) * _NW + w) < _NCHUNK)
                def _(b2=(j - 1) % 2):
                    pltpu.make_async_copy(rows.at[b2], accum.at[didx.at[b2]],
                                          ssem.at[b2]).wait()

            if j < _MAXJ:
                fire_idx(j + 1)

                @pl.when(cid < _NCHUNK)
                def _(b=b):
                    pltpu.async_copy(rows.at[b], accum.at[didx.at[b]],
                                     ssem.at[b], add=True)
        plsc.subcore_barrier()
        pltpu.sync_copy(accum.at[pl.ds(row0, _RPT), :],
                        out_hbm.at[c, pl.ds(row0, _RPT), :])

    return body


@functools.cache
def _sc_aggr_kernel(linear=False):
    return pl.kernel(
        _make_sc_body(linear),
        out_type=jax.ShapeDtypeStruct((_NC, _NPAD, _DF), jnp.float32),
        mesh=plsc.VectorSubcoreMesh(core_axis_name="c", subcore_axis_name="s"),
        scratch_types=[
            pltpu.VMEM((_K, _CH), jnp.int32),
            pltpu.VMEM((_K, _CH), jnp.int32),
            pltpu.VMEM((_K, _CH, _DF), jnp.float32),
            pltpu.VMEM_SHARED((_NPAD, _DF), jnp.float32),
            pltpu.SemaphoreType.DMA((4,)),
            pltpu.SemaphoreType.DMA((_K,)),
            pltpu.SemaphoreType.DMA((_K,)),
        ],
    )


_BN = 1000
_EB = 4000


def _relu_body(x_ref, gx_ref):
    gx_ref[...] = jnp.maximum(x_ref[...], 0.0)


def _relu_x(x):
    return pl.pallas_call(
        _relu_body,
        grid=(_N // _BN,),
        in_specs=[pl.BlockSpec((_BN, _DF), lambda i: (i, 0))],
        out_specs=pl.BlockSpec((_BN, _DF), lambda i: (i, 0)),
        out_shape=jax.ShapeDtypeStruct((_N, _DF), jnp.float32),
    )(x)


def _eapad_body(ea_ref, eap_ref):
    eap_ref[...] = jnp.concatenate(
        [jnp.maximum(ea_ref[...], 0.0),
         jnp.zeros((_EB, _DF - _DE), jnp.float32)], axis=1)


def _eapad(ea):
    return pl.pallas_call(
        _eapad_body,
        grid=(_E // _EB,),
        in_specs=[pl.BlockSpec((_EB, _DE), lambda i: (i, 0))],
        out_specs=pl.BlockSpec((_EB, _DF), lambda i: (i, 0)),
        out_shape=jax.ShapeDtypeStruct((_E, _DF), jnp.float32),
    )(ea)


def _layer_body(h_ref, p_ref, es_ref, W1a_ref, W1b_ref, b1_ref, W2_ref,
                b2_ref, o_ref):
    z = h_ref[...] + p_ref[0] + p_ref[1]
    es = es_ref[0] + es_ref[1]
    t = jnp.dot(z, W1a_ref[...], preferred_element_type=jnp.float32)
    t = t + jnp.dot(es, W1b_ref[...], preferred_element_type=jnp.float32)
    t = jnp.maximum(t + b1_ref[...], 0.0)
    o = jnp.dot(t, W2_ref[...], preferred_element_type=jnp.float32)
    o_ref[...] = jnp.maximum(o + b2_ref[...], 0.0)


def _layer(h, p, es, W1a, W1b, b1, W2, b2):
    return pl.pallas_call(
        _layer_body,
        grid=(_N // _BN,),
        in_specs=[
            pl.BlockSpec((_BN, _DF), lambda i: (i, 0)),
            pl.BlockSpec((_NC, _BN, _DF), lambda i: (0, i, 0)),
            pl.BlockSpec((_NC, _BN, _DF), lambda i: (0, i, 0)),
            pl.BlockSpec((_H, _H), lambda i: (0, 0)),
            pl.BlockSpec((_DF, _H), lambda i: (0, 0)),
            pl.BlockSpec((1, _H), lambda i: (0, 0)),
            pl.BlockSpec((_H, _H), lambda i: (0, 0)),
            pl.BlockSpec((1, _H), lambda i: (0, 0)),
        ],
        out_specs=pl.BlockSpec((_BN, _H), lambda i: (i, 0)),
        out_shape=jax.ShapeDtypeStruct((_N, _H), jnp.float32),
    )(h, p, es, W1a, W1b, b1, W2, b2)


def _pool_body(h_ref, b_ref, Wl_ref, bl_ref, o_ref, acc, cnt):
    i = pl.program_id(0)

    @pl.when(i == 0)
    def _():
        acc[...] = jnp.zeros_like(acc)
        cnt[...] = jnp.zeros_like(cnt)

    bvec = b_ref[0, 0]
    onehot = (bvec[:, None]
              == lax.broadcasted_iota(jnp.int32, (1, _G), 1)).astype(jnp.float32)
    acc[...] += lax.dot_general(onehot, h_ref[...], (((0,), (0,)), ((), ())),
                                preferred_element_type=jnp.float32)
    ones = jnp.ones((_BN, _H), jnp.float32)
    cnt[...] += lax.dot_general(onehot, ones, (((0,), (0,)), ((), ())),
                                preferred_element_type=jnp.float32)

    @pl.when(i == _N // _BN - 1)
    def _():
        pooled = acc[...] / jnp.maximum(cnt[...], 1.0)
        o_ref[...] = (jnp.dot(pooled, Wl_ref[...],
                              preferred_element_type=jnp.float32)
                      + bl_ref[...])


def _pool(h, batch3d, W_lin, b_lin2):
    return pl.pallas_call(
        _pool_body,
        grid=(_N // _BN,),
        in_specs=[
            pl.BlockSpec((_BN, _H), lambda i: (i, 0)),
            pl.BlockSpec((1, 1, _BN), lambda i: (i, 0, 0)),
            pl.BlockSpec((_H, _OUT), lambda i: (0, 0)),
            pl.BlockSpec((1, _OUT), lambda i: (0, 0)),
        ],
        out_specs=pl.BlockSpec((_G, _OUT), lambda i: (0, 0)),
        out_shape=jax.ShapeDtypeStruct((_G, _OUT), jnp.float32),
        scratch_shapes=[
            pltpu.VMEM((_G, _H), jnp.float32),
            pltpu.VMEM((_G, _H), jnp.float32),
        ],
    )(h, batch3d, W_lin, b_lin2)


def kernel(x, edge_index, edge_attr, batch,
           W1_0, b1_0, W2_0, b2_0,
           W1_1, b1_1, W2_1, b2_1,
           W1_2, b1_2, W2_2, b2_2,
           W_lin, b_lin):
    g0 = _relu_x(x)
    ea_pad = _eapad(edge_attr)

    z128 = jnp.zeros((_RPT, _DF), jnp.float32)
    src = edge_index[0]
    dst = edge_index[1]

    es = _sc_aggr_kernel(True)(ea_pad, dst, dst, z128)

    h = x
    g = g0
    for (W1, b1, W2, b2) in ((W1_0, b1_0, W2_0, b2_0),
                             (W1_1, b1_1, W2_1, b2_1),
                             (W1_2, b1_2, W2_2, b2_2)):
        p = _sc_aggr_kernel()(g, src, dst, z128)
        W1b_pad = jnp.concatenate(
            [W1[_H:], jnp.zeros((_DF - _DE, _H), jnp.float32)], axis=0)
        h = _layer(h, p, es, W1[:_H], W1b_pad, b1.reshape(1, _H),
                   W2, b2.reshape(1, _H))
        g = h

    batch3d = batch.reshape(_N // _BN, 1, _BN)
    return _pool(h, batch3d, W_lin, b_lin.reshape(1, _OUT))


# speedup vs baseline: 1.0114x; 1.0094x over previous
import functools

import jax
import jax.numpy as jnp
from jax import lax
from jax.experimental import pallas as pl
from jax.experimental.pallas import tpu as pltpu
from jax.experimental.pallas import tpu_sc as plsc

_N, _E, _DF, _DE, _H, _OUT, _G = 10000, 320000, 128, 16, 128, 64, 64
_NC, _NS = 2, 16
_NW = _NC * _NS
_CH = 64
_NCHUNK = _E // _CH
_MAXJ = -(-_NCHUNK // _NW)
_NPAD = 10240
_RPT = _NPAD // _NS

_K = 4


def _make_sc_body(linear):
    IC = 4

    def body(g_hbm, src_hbm, dst_hbm, z_hbm, out_hbm, sidx, didx, rows,
             accum, isem, gsem, ssem):
        c = lax.axis_index("c")
        s = lax.axis_index("s")
        w = s * _NC + c
        row0 = s * _RPT
        pltpu.sync_copy(z_hbm, accum.at[pl.ds(row0, _RPT), :])
        plsc.subcore_barrier()

        n_groups = -(-_MAXJ // _K)

        @pl.loop(0, n_groups)
        def group(gi):
            cids = [(gi * _K + b) * _NW + w for b in range(_K)]

            for b in range(_K):
                @pl.when(cids[b] < _NCHUNK)
                def _(b=b, cid=cids[b]):
                    base = cid * _CH
                    if not linear:
                        pltpu.async_copy(src_hbm.at[pl.ds(base, _CH)],
                                         sidx.at[b], isem.at[b])
                    pltpu.async_copy(dst_hbm.at[pl.ds(base, _CH)],
                                     didx.at[b], isem.at[b])

            for b in range(_K):
                @pl.when(cids[b] < _NCHUNK)
                def _(b=b, cid=cids[b]):
                    if not linear:
                        pltpu.make_async_copy(src_hbm.at[pl.ds(0, _CH)],
                                              sidx.at[b], isem.at[b]).wait()
                    pltpu.make_async_copy(dst_hbm.at[pl.ds(0, _CH)],
                                          didx.at[b], isem.at[b]).wait()
                    if linear:
                        base = cid * _CH
                        pltpu.async_copy(g_hbm.at[pl.ds(base, _CH), :],
                                         rows.at[b], gsem.at[b])
                    else:
                        pltpu.async_copy(g_hbm.at[sidx.at[b]], rows.at[b],
                                         gsem.at[b])

            for b in range(_K):
                @pl.when(cids[b] < _NCHUNK)
                def _(b=b, cid=cids[b]):
                    if linear:
                        pltpu.make_async_copy(g_hbm.at[pl.ds(0, _CH), :],
                                              rows.at[b], gsem.at[b]).wait()
                    else:
                        pltpu.make_async_copy(g_hbm.at[sidx.at[b]],
                                              rows.at[b], gsem.at[b]).wait()

            for b in range(_K):
                @pl.when(cids[b] < _NCHUNK)
                def _(b=b):
                    pltpu.async_copy(rows.at[b], accum.at[didx.at[b]],
                                     ssem.at[b], add=True)

            for b in range(_K):
                @pl.when(cids[b] < _NCHUNK)
                def _(b=b):
                    pltpu.make_async_copy(rows.at[b], accum.at[didx.at[b]],
                                          ssem.at[b]).wait()
        plsc.subcore_barrier()
        pltpu.sync_copy(accum.at[pl.ds(row0, _RPT), :],
                        out_hbm.at[c, pl.ds(row0, _RPT), :])

    return body


@functools.cache
def _sc_aggr_kernel(linear=False):
    return pl.kernel(
        _make_sc_body(linear),
        out_type=jax.ShapeDtypeStruct((_NC, _NPAD, _DF), jnp.float32),
        mesh=plsc.VectorSubcoreMesh(core_axis_name="c", subcore_axis_name="s"),
        scratch_types=[
            pltpu.VMEM((_K, _CH), jnp.int32),
            pltpu.VMEM((_K, _CH), jnp.int32),
            pltpu.VMEM((_K, _CH, _DF), jnp.float32),
            pltpu.VMEM_SHARED((_NPAD, _DF), jnp.float32),
            pltpu.SemaphoreType.DMA((_K,)),
            pltpu.SemaphoreType.DMA((_K,)),
            pltpu.SemaphoreType.DMA((_K,)),
        ],
    )


_BN = 1000
_EB = 4000


def _relu_body(x_ref, gx_ref):
    gx_ref[...] = jnp.maximum(x_ref[...], 0.0)


def _relu_x(x):
    return pl.pallas_call(
        _relu_body,
        grid=(_N // _BN,),
        in_specs=[pl.BlockSpec((_BN, _DF), lambda i: (i, 0))],
        out_specs=pl.BlockSpec((_BN, _DF), lambda i: (i, 0)),
        out_shape=jax.ShapeDtypeStruct((_N, _DF), jnp.float32),
    )(x)


def _eapad_body(ea_ref, eap_ref):
    eap_ref[...] = jnp.concatenate(
        [jnp.maximum(ea_ref[...], 0.0),
         jnp.zeros((_EB, _DF - _DE), jnp.float32)], axis=1)


def _eapad(ea):
    return pl.pallas_call(
        _eapad_body,
        grid=(_E // _EB,),
        in_specs=[pl.BlockSpec((_EB, _DE), lambda i: (i, 0))],
        out_specs=pl.BlockSpec((_EB, _DF), lambda i: (i, 0)),
        out_shape=jax.ShapeDtypeStruct((_E, _DF), jnp.float32),
    )(ea)


def _layer_body(h_ref, p_ref, es_ref, W1a_ref, W1b_ref, b1_ref, W2_ref,
                b2_ref, o_ref):
    z = h_ref[...] + p_ref[0] + p_ref[1]
    es = es_ref[0] + es_ref[1]
    t = jnp.dot(z, W1a_ref[...], preferred_element_type=jnp.float32)
    t = t + jnp.dot(es, W1b_ref[...], preferred_element_type=jnp.float32)
    t = jnp.maximum(t + b1_ref[...], 0.0)
    o = jnp.dot(t, W2_ref[...], preferred_element_type=jnp.float32)
    o_ref[...] = jnp.maximum(o + b2_ref[...], 0.0)


def _layer(h, p, es, W1a, W1b, b1, W2, b2):
    return pl.pallas_call(
        _layer_body,
        grid=(_N // _BN,),
        in_specs=[
            pl.BlockSpec((_BN, _DF), lambda i: (i, 0)),
            pl.BlockSpec((_NC, _BN, _DF), lambda i: (0, i, 0)),
            pl.BlockSpec((_NC, _BN, _DF), lambda i: (0, i, 0)),
            pl.BlockSpec((_H, _H), lambda i: (0, 0)),
            pl.BlockSpec((_DF, _H), lambda i: (0, 0)),
            pl.BlockSpec((1, _H), lambda i: (0, 0)),
            pl.BlockSpec((_H, _H), lambda i: (0, 0)),
            pl.BlockSpec((1, _H), lambda i: (0, 0)),
        ],
        out_specs=pl.BlockSpec((_BN, _H), lambda i: (i, 0)),
        out_shape=jax.ShapeDtypeStruct((_N, _H), jnp.float32),
    )(h, p, es, W1a, W1b, b1, W2, b2)


def _pool_body(h_ref, b_ref, Wl_ref, bl_ref, o_ref, acc, cnt):
    i = pl.program_id(0)

    @pl.when(i == 0)
    def _():
        acc[...] = jnp.zeros_like(acc)
        cnt[...] = jnp.zeros_like(cnt)

    bvec = b_ref[0, 0]
    onehot = (bvec[:, None]
              == lax.broadcasted_iota(jnp.int32, (1, _G), 1)).astype(jnp.float32)
    acc[...] += lax.dot_general(onehot, h_ref[...], (((0,), (0,)), ((), ())),
                                preferred_element_type=jnp.float32)
    ones = jnp.ones((_BN, _H), jnp.float32)
    cnt[...] += lax.dot_general(onehot, ones, (((0,), (0,)), ((), ())),
                                preferred_element_type=jnp.float32)

    @pl.when(i == _N // _BN - 1)
    def _():
        pooled = acc[...] / jnp.maximum(cnt[...], 1.0)
        o_ref[...] = (jnp.dot(pooled, Wl_ref[...],
                              preferred_element_type=jnp.float32)
                      + bl_ref[...])


def _pool(h, batch3d, W_lin, b_lin2):
    return pl.pallas_call(
        _pool_body,
        grid=(_N // _BN,),
        in_specs=[
            pl.BlockSpec((_BN, _H), lambda i: (i, 0)),
            pl.BlockSpec((1, 1, _BN), lambda i: (i, 0, 0)),
            pl.BlockSpec((_H, _OUT), lambda i: (0, 0)),
            pl.BlockSpec((1, _OUT), lambda i: (0, 0)),
        ],
        out_specs=pl.BlockSpec((_G, _OUT), lambda i: (0, 0)),
        out_shape=jax.ShapeDtypeStruct((_G, _OUT), jnp.float32),
        scratch_shapes=[
            pltpu.VMEM((_G, _H), jnp.float32),
            pltpu.VMEM((_G, _H), jnp.float32),
        ],
    )(h, batch3d, W_lin, b_lin2)


def kernel(x, edge_index, edge_attr, batch,
           W1_0, b1_0, W2_0, b2_0,
           W1_1, b1_1, W2_1, b2_1,
           W1_2, b1_2, W2_2, b2_2,
           W_lin, b_lin):
    g0 = _relu_x(x)
    ea_pad = _eapad(edge_attr)

    z128 = jnp.zeros((_RPT, _DF), jnp.float32)
    src = edge_index[0]
    dst = edge_index[1]

    es = _sc_aggr_kernel(True)(ea_pad, dst, dst, z128)

    h = x
    g = g0
    for (W1, b1, W2, b2) in ((W1_0, b1_0, W2_0, b2_0),
                             (W1_1, b1_1, W2_1, b2_1),
                             (W1_2, b1_2, W2_2, b2_2)):
        p = _sc_aggr_kernel()(g, src, dst, z128)
        W1b_pad = jnp.concatenate(
            [W1[_H:], jnp.zeros((_DF - _DE, _H), jnp.float32)], axis=0)
        h = _layer(h, p, es, W1[:_H], W1b_pad, b1.reshape(1, _H),
                   W2, b2.reshape(1, _H))
        g = h

    batch3d = batch.reshape(_N // _BN, 1, _BN)
    return _pool(h, batch3d, W_lin, b_lin.reshape(1, _OUT))


# speedup vs baseline: 1.0193x; 1.0078x over previous
import functools

import jax
import jax.numpy as jnp
from jax import lax
from jax.experimental import pallas as pl
from jax.experimental.pallas import tpu as pltpu
from jax.experimental.pallas import tpu_sc as plsc

_N, _E, _DF, _DE, _H, _OUT, _G = 10000, 320000, 128, 16, 128, 64, 64
_NC, _NS = 2, 16
_NW = _NC * _NS
_CH = 64
_NCHUNK = _E // _CH
_MAXJ = -(-_NCHUNK // _NW)
_NPAD = 10240
_RPT = _NPAD // _NS

_K = 4


def _make_sc_body(linear):
    IC = 4

    def body(g_hbm, src_hbm, dst_hbm, z_hbm, out_hbm, sidx, didx, rows,
             accum, isem, gsem, ssem):
        c = lax.axis_index("c")
        s = lax.axis_index("s")
        w = s * _NC + c
        row0 = s * _RPT
        pltpu.sync_copy(z_hbm, accum.at[pl.ds(row0, _RPT), :])
        plsc.subcore_barrier()

        n_groups = -(-_MAXJ // _K)

        @pl.loop(0, n_groups)
        def group(gi):
            cids = [(gi * _K + b) * _NW + w for b in range(_K)]

            for b in range(_K):
                @pl.when(cids[b] < _NCHUNK)
                def _(b=b, cid=cids[b]):
                    base = cid * _CH
                    if not linear:
                        pltpu.async_copy(src_hbm.at[pl.ds(base, _CH)],
                                         sidx.at[b], isem)
                    pltpu.async_copy(dst_hbm.at[pl.ds(base, _CH)],
                                     didx.at[b], isem)

            for b in range(_K):
                @pl.when(cids[b] < _NCHUNK)
                def _(b=b, cid=cids[b]):
                    if not linear:
                        pltpu.make_async_copy(src_hbm.at[pl.ds(0, _CH)],
                                              sidx.at[b], isem).wait()
                    pltpu.make_async_copy(dst_hbm.at[pl.ds(0, _CH)],
                                          didx.at[b], isem).wait()
                    if linear:
                        base = cid * _CH
                        pltpu.async_copy(g_hbm.at[pl.ds(base, _CH), :],
                                         rows.at[b], gsem)
                    else:
                        pltpu.async_copy(g_hbm.at[sidx.at[b]], rows.at[b],
                                         gsem)

            for b in range(_K):
                @pl.when(cids[b] < _NCHUNK)
                def _(b=b, cid=cids[b]):
                    if linear:
                        pltpu.make_async_copy(g_hbm.at[pl.ds(0, _CH), :],
                                              rows.at[b], gsem).wait()
                    else:
                        pltpu.make_async_copy(g_hbm.at[sidx.at[b]],
                                              rows.at[b], gsem).wait()

            for b in range(_K):
                @pl.when(cids[b] < _NCHUNK)
                def _(b=b):
                    pltpu.async_copy(rows.at[b], accum.at[didx.at[b]],
                                     ssem, add=True)

            for b in range(_K):
                @pl.when(cids[b] < _NCHUNK)
                def _(b=b):
                    pltpu.make_async_copy(rows.at[b], accum.at[didx.at[b]],
                                          ssem).wait()
        plsc.subcore_barrier()
        pltpu.sync_copy(accum.at[pl.ds(row0, _RPT), :],
                        out_hbm.at[c, pl.ds(row0, _RPT), :])

    return body


@functools.cache
def _sc_aggr_kernel(linear=False):
    return pl.kernel(
        _make_sc_body(linear),
        out_type=jax.ShapeDtypeStruct((_NC, _NPAD, _DF), jnp.float32),
        mesh=plsc.VectorSubcoreMesh(core_axis_name="c", subcore_axis_name="s"),
        scratch_types=[
            pltpu.VMEM((_K, _CH), jnp.int32),
            pltpu.VMEM((_K, _CH), jnp.int32),
            pltpu.VMEM((_K, _CH, _DF), jnp.float32),
            pltpu.VMEM_SHARED((_NPAD, _DF), jnp.float32),
            pltpu.SemaphoreType.DMA,
            pltpu.SemaphoreType.DMA,
            pltpu.SemaphoreType.DMA,
        ],
    )


_BN = 1000
_EB = 4000


def _relu_body(x_ref, gx_ref):
    gx_ref[...] = jnp.maximum(x_ref[...], 0.0)


def _relu_x(x):
    return pl.pallas_call(
        _relu_body,
        grid=(_N // _BN,),
        in_specs=[pl.BlockSpec((_BN, _DF), lambda i: (i, 0))],
        out_specs=pl.BlockSpec((_BN, _DF), lambda i: (i, 0)),
        out_shape=jax.ShapeDtypeStruct((_N, _DF), jnp.float32),
    )(x)


def _eapad_body(ea_ref, eap_ref):
    eap_ref[...] = jnp.concatenate(
        [jnp.maximum(ea_ref[...], 0.0),
         jnp.zeros((_EB, _DF - _DE), jnp.float32)], axis=1)


def _eapad(ea):
    return pl.pallas_call(
        _eapad_body,
        grid=(_E // _EB,),
        in_specs=[pl.BlockSpec((_EB, _DE), lambda i: (i, 0))],
        out_specs=pl.BlockSpec((_EB, _DF), lambda i: (i, 0)),
        out_shape=jax.ShapeDtypeStruct((_E, _DF), jnp.float32),
    )(ea)


def _layer_body(h_ref, p_ref, es_ref, W1a_ref, W1b_ref, b1_ref, W2_ref,
                b2_ref, o_ref):
    z = h_ref[...] + p_ref[0] + p_ref[1]
    es = es_ref[0] + es_ref[1]
    t = jnp.dot(z, W1a_ref[...], preferred_element_type=jnp.float32)
    t = t + jnp.dot(es, W1b_ref[...], preferred_element_type=jnp.float32)
    t = jnp.maximum(t + b1_ref[...], 0.0)
    o = jnp.dot(t, W2_ref[...], preferred_element_type=jnp.float32)
    o_ref[...] = jnp.maximum(o + b2_ref[...], 0.0)


def _layer(h, p, es, W1a, W1b, b1, W2, b2):
    return pl.pallas_call(
        _layer_body,
        grid=(_N // _BN,),
        in_specs=[
            pl.BlockSpec((_BN, _DF), lambda i: (i, 0)),
            pl.BlockSpec((_NC, _BN, _DF), lambda i: (0, i, 0)),
            pl.BlockSpec((_NC, _BN, _DF), lambda i: (0, i, 0)),
            pl.BlockSpec((_H, _H), lambda i: (0, 0)),
            pl.BlockSpec((_DF, _H), lambda i: (0, 0)),
            pl.BlockSpec((1, _H), lambda i: (0, 0)),
            pl.BlockSpec((_H, _H), lambda i: (0, 0)),
            pl.BlockSpec((1, _H), lambda i: (0, 0)),
        ],
        out_specs=pl.BlockSpec((_BN, _H), lambda i: (i, 0)),
        out_shape=jax.ShapeDtypeStruct((_N, _H), jnp.float32),
    )(h, p, es, W1a, W1b, b1, W2, b2)


def _pool_body(h_ref, b_ref, Wl_ref, bl_ref, o_ref, acc, cnt):
    i = pl.program_id(0)

    @pl.when(i == 0)
    def _():
        acc[...] = jnp.zeros_like(acc)
        cnt[...] = jnp.zeros_like(cnt)

    bvec = b_ref[0, 0]
    onehot = (bvec[:, None]
              == lax.broadcasted_iota(jnp.int32, (1, _G), 1)).astype(jnp.float32)
    acc[...] += lax.dot_general(onehot, h_ref[...], (((0,), (0,)), ((), ())),
                                preferred_element_type=jnp.float32)
    ones = jnp.ones((_BN, _H), jnp.float32)
    cnt[...] += lax.dot_general(onehot, ones, (((0,), (0,)), ((), ())),
                                preferred_element_type=jnp.float32)

    @pl.when(i == _N // _BN - 1)
    def _():
        pooled = acc[...] / jnp.maximum(cnt[...], 1.0)
        o_ref[...] = (jnp.dot(pooled, Wl_ref[...],
                              preferred_element_type=jnp.float32)
                      + bl_ref[...])


def _pool(h, batch3d, W_lin, b_lin2):
    return pl.pallas_call(
        _pool_body,
        grid=(_N // _BN,),
        in_specs=[
            pl.BlockSpec((_BN, _H), lambda i: (i, 0)),
            pl.BlockSpec((1, 1, _BN), lambda i: (i, 0, 0)),
            pl.BlockSpec((_H, _OUT), lambda i: (0, 0)),
            pl.BlockSpec((1, _OUT), lambda i: (0, 0)),
        ],
        out_specs=pl.BlockSpec((_G, _OUT), lambda i: (0, 0)),
        out_shape=jax.ShapeDtypeStruct((_G, _OUT), jnp.float32),
        scratch_shapes=[
            pltpu.VMEM((_G, _H), jnp.float32),
            pltpu.VMEM((_G, _H), jnp.float32),
        ],
    )(h, batch3d, W_lin, b_lin2)


def kernel(x, edge_index, edge_attr, batch,
           W1_0, b1_0, W2_0, b2_0,
           W1_1, b1_1, W2_1, b2_1,
           W1_2, b1_2, W2_2, b2_2,
           W_lin, b_lin):
    g0 = _relu_x(x)
    ea_pad = _eapad(edge_attr)

    z128 = jnp.zeros((_RPT, _DF), jnp.float32)
    src = edge_index[0]
    dst = edge_index[1]

    es = _sc_aggr_kernel(True)(ea_pad, dst, dst, z128)

    h = x
    g = g0
    for (W1, b1, W2, b2) in ((W1_0, b1_0, W2_0, b2_0),
                             (W1_1, b1_1, W2_1, b2_1),
                             (W1_2, b1_2, W2_2, b2_2)):
        p = _sc_aggr_kernel()(g, src, dst, z128)
        W1b_pad = jnp.concatenate(
            [W1[_H:], jnp.zeros((_DF - _DE, _H), jnp.float32)], axis=0)
        h = _layer(h, p, es, W1[:_H], W1b_pad, b1.reshape(1, _H),
                   W2, b2.reshape(1, _H))
        g = h

    batch3d = batch.reshape(_N // _BN, 1, _BN)
    return _pool(h, batch3d, W_lin, b_lin.reshape(1, _OUT))


# speedup vs baseline: 1.0936x; 1.0729x over previous
import functools

import jax
import jax.numpy as jnp
from jax import lax
from jax.experimental import pallas as pl
from jax.experimental.pallas import tpu as pltpu
from jax.experimental.pallas import tpu_sc as plsc

_N, _E, _DF, _DE, _H, _OUT, _G = 10000, 320000, 128, 16, 128, 64, 64
_NC, _NS = 2, 16
_NW = _NC * _NS
_CH = 128
_NCHUNK = _E // _CH
_MAXJ = -(-_NCHUNK // _NW)
_NPAD = 10240
_RPT = _NPAD // _NS

_K = 2


def _make_sc_body(linear):
    IC = 4

    def body(g_hbm, src_hbm, dst_hbm, z_hbm, out_hbm, sidx, didx, rows,
             accum, isem, gsem, ssem):
        c = lax.axis_index("c")
        s = lax.axis_index("s")
        w = s * _NC + c
        row0 = s * _RPT
        pltpu.sync_copy(z_hbm, accum.at[pl.ds(row0, _RPT), :])
        plsc.subcore_barrier()

        n_groups = -(-_MAXJ // _K)

        @pl.loop(0, n_groups)
        def group(gi):
            cids = [(gi * _K + b) * _NW + w for b in range(_K)]

            for b in range(_K):
                @pl.when(cids[b] < _NCHUNK)
                def _(b=b, cid=cids[b]):
                    base = cid * _CH
                    if not linear:
                        pltpu.async_copy(src_hbm.at[pl.ds(base, _CH)],
                                         sidx.at[b], isem)
                    pltpu.async_copy(dst_hbm.at[pl.ds(base, _CH)],
                                     didx.at[b], isem)

            for b in range(_K):
                @pl.when(cids[b] < _NCHUNK)
                def _(b=b, cid=cids[b]):
                    if not linear:
                        pltpu.make_async_copy(src_hbm.at[pl.ds(0, _CH)],
                                              sidx.at[b], isem).wait()
                    pltpu.make_async_copy(dst_hbm.at[pl.ds(0, _CH)],
                                          didx.at[b], isem).wait()
                    if linear:
                        base = cid * _CH
                        pltpu.async_copy(g_hbm.at[pl.ds(base, _CH), :],
                                         rows.at[b], gsem)
                    else:
                        pltpu.async_copy(g_hbm.at[sidx.at[b]], rows.at[b],
                                         gsem)

            for b in range(_K):
                @pl.when(cids[b] < _NCHUNK)
                def _(b=b, cid=cids[b]):
                    if linear:
                        pltpu.make_async_copy(g_hbm.at[pl.ds(0, _CH), :],
                                              rows.at[b], gsem).wait()
                    else:
                        pltpu.make_async_copy(g_hbm.at[sidx.at[b]],
                                              rows.at[b], gsem).wait()
                    pltpu.async_copy(rows.at[b], accum.at[didx.at[b]],
                                     ssem, add=True)

            for b in range(_K):
                @pl.when(cids[b] < _NCHUNK)
                def _(b=b):
                    pltpu.make_async_copy(rows.at[b], accum.at[didx.at[b]],
                                          ssem).wait()
        plsc.subcore_barrier()
        pltpu.sync_copy(accum.at[pl.ds(row0, _RPT), :],
                        out_hbm.at[c, pl.ds(row0, _RPT), :])

    return body


@functools.cache
def _sc_aggr_kernel(linear=False):
    return pl.kernel(
        _make_sc_body(linear),
        out_type=jax.ShapeDtypeStruct((_NC, _NPAD, _DF), jnp.float32),
        mesh=plsc.VectorSubcoreMesh(core_axis_name="c", subcore_axis_name="s"),
        scratch_types=[
            pltpu.VMEM((_K, _CH), jnp.int32),
            pltpu.VMEM((_K, _CH), jnp.int32),
            pltpu.VMEM((_K, _CH, _DF), jnp.float32),
            pltpu.VMEM_SHARED((_NPAD, _DF), jnp.float32),
            pltpu.SemaphoreType.DMA,
            pltpu.SemaphoreType.DMA,
            pltpu.SemaphoreType.DMA,
        ],
    )


_BN = 1000
_EB = 4000


def _relu_body(x_ref, gx_ref):
    gx_ref[...] = jnp.maximum(x_ref[...], 0.0)


def _relu_x(x):
    return pl.pallas_call(
        _relu_body,
        grid=(_N // _BN,),
        in_specs=[pl.BlockSpec((_BN, _DF), lambda i: (i, 0))],
        out_specs=pl.BlockSpec((_BN, _DF), lambda i: (i, 0)),
        out_shape=jax.ShapeDtypeStruct((_N, _DF), jnp.float32),
    )(x)


def _eapad_body(ea_ref, eap_ref):
    eap_ref[...] = jnp.concatenate(
        [jnp.maximum(ea_ref[...], 0.0),
         jnp.zeros((_EB, _DF - _DE), jnp.float32)], axis=1)


def _eapad(ea):
    return pl.pallas_call(
        _eapad_body,
        grid=(_E // _EB,),
        in_specs=[pl.BlockSpec((_EB, _DE), lambda i: (i, 0))],
        out_specs=pl.BlockSpec((_EB, _DF), lambda i: (i, 0)),
        out_shape=jax.ShapeDtypeStruct((_E, _DF), jnp.float32),
    )(ea)


def _layer_body(h_ref, p_ref, es_ref, W1a_ref, W1b_ref, b1_ref, W2_ref,
                b2_ref, o_ref):
    z = h_ref[...] + p_ref[0] + p_ref[1]
    es = es_ref[0] + es_ref[1]
    t = jnp.dot(z, W1a_ref[...], preferred_element_type=jnp.float32)
    t = t + jnp.dot(es, W1b_ref[...], preferred_element_type=jnp.float32)
    t = jnp.maximum(t + b1_ref[...], 0.0)
    o = jnp.dot(t, W2_ref[...], preferred_element_type=jnp.float32)
    o_ref[...] = jnp.maximum(o + b2_ref[...], 0.0)


def _layer(h, p, es, W1a, W1b, b1, W2, b2):
    return pl.pallas_call(
        _layer_body,
        grid=(_N // _BN,),
        in_specs=[
            pl.BlockSpec((_BN, _DF), lambda i: (i, 0)),
            pl.BlockSpec((_NC, _BN, _DF), lambda i: (0, i, 0)),
            pl.BlockSpec((_NC, _BN, _DF), lambda i: (0, i, 0)),
            pl.BlockSpec((_H, _H), lambda i: (0, 0)),
            pl.BlockSpec((_DF, _H), lambda i: (0, 0)),
            pl.BlockSpec((1, _H), lambda i: (0, 0)),
            pl.BlockSpec((_H, _H), lambda i: (0, 0)),
            pl.BlockSpec((1, _H), lambda i: (0, 0)),
        ],
        out_specs=pl.BlockSpec((_BN, _H), lambda i: (i, 0)),
        out_shape=jax.ShapeDtypeStruct((_N, _H), jnp.float32),
    )(h, p, es, W1a, W1b, b1, W2, b2)


def _pool_body(h_ref, b_ref, Wl_ref, bl_ref, o_ref, acc, cnt):
    i = pl.program_id(0)

    @pl.when(i == 0)
    def _():
        acc[...] = jnp.zeros_like(acc)
        cnt[...] = jnp.zeros_like(cnt)

    bvec = b_ref[0, 0]
    onehot = (bvec[:, None]
              == lax.broadcasted_iota(jnp.int32, (1, _G), 1)).astype(jnp.float32)
    acc[...] += lax.dot_general(onehot, h_ref[...], (((0,), (0,)), ((), ())),
                                preferred_element_type=jnp.float32)
    ones = jnp.ones((_BN, _H), jnp.float32)
    cnt[...] += lax.dot_general(onehot, ones, (((0,), (0,)), ((), ())),
                                preferred_element_type=jnp.float32)

    @pl.when(i == _N // _BN - 1)
    def _():
        pooled = acc[...] / jnp.maximum(cnt[...], 1.0)
        o_ref[...] = (jnp.dot(pooled, Wl_ref[...],
                              preferred_element_type=jnp.float32)
                      + bl_ref[...])


def _pool(h, batch3d, W_lin, b_lin2):
    return pl.pallas_call(
        _pool_body,
        grid=(_N // _BN,),
        in_specs=[
            pl.BlockSpec((_BN, _H), lambda i: (i, 0)),
            pl.BlockSpec((1, 1, _BN), lambda i: (i, 0, 0)),
            pl.BlockSpec((_H, _OUT), lambda i: (0, 0)),
            pl.BlockSpec((1, _OUT), lambda i: (0, 0)),
        ],
        out_specs=pl.BlockSpec((_G, _OUT), lambda i: (0, 0)),
        out_shape=jax.ShapeDtypeStruct((_G, _OUT), jnp.float32),
        scratch_shapes=[
            pltpu.VMEM((_G, _H), jnp.float32),
            pltpu.VMEM((_G, _H), jnp.float32),
        ],
    )(h, batch3d, W_lin, b_lin2)


def kernel(x, edge_index, edge_attr, batch,
           W1_0, b1_0, W2_0, b2_0,
           W1_1, b1_1, W2_1, b2_1,
           W1_2, b1_2, W2_2, b2_2,
           W_lin, b_lin):
    g0 = _relu_x(x)
    ea_pad = _eapad(edge_attr)

    z128 = jnp.zeros((_RPT, _DF), jnp.float32)
    src = edge_index[0]
    dst = edge_index[1]

    es = _sc_aggr_kernel(True)(ea_pad, dst, dst, z128)

    h = x
    g = g0
    for (W1, b1, W2, b2) in ((W1_0, b1_0, W2_0, b2_0),
                             (W1_1, b1_1, W2_1, b2_1),
                             (W1_2, b1_2, W2_2, b2_2)):
        p = _sc_aggr_kernel()(g, src, dst, z128)
        W1b_pad = jnp.concatenate(
            [W1[_H:], jnp.zeros((_DF - _DE, _H), jnp.float32)], axis=0)
        h = _layer(h, p, es, W1[:_H], W1b_pad, b1.reshape(1, _H),
                   W2, b2.reshape(1, _H))
        g = h

    batch3d = batch.reshape(_N // _BN, 1, _BN)
    return _pool(h, batch3d, W_lin, b_lin.reshape(1, _OUT))


# speedup vs baseline: 1.1566x; 1.0576x over previous
import functools

import jax
import jax.numpy as jnp
from jax import lax
from jax.experimental import pallas as pl
from jax.experimental.pallas import tpu as pltpu
from jax.experimental.pallas import tpu_sc as plsc

_N, _E, _DF, _DE, _H, _OUT, _G = 10000, 320000, 128, 16, 128, 64, 64
_NC, _NS = 2, 16
_NW = _NC * _NS
_CH = 128
_NCHUNK = _E // _CH
_MAXJ = -(-_NCHUNK // _NW)
_NPAD = 10112
_RPT = _NPAD // _NS

_K = 3


def _make_sc_body(linear):
    IC = 4

    def body(g_hbm, src_hbm, dst_hbm, z_hbm, out_hbm, sidx, didx, rows,
             accum, isem, gsem, ssem):
        c = lax.axis_index("c")
        s = lax.axis_index("s")
        w = s * _NC + c
        row0 = s * _RPT
        pltpu.sync_copy(z_hbm, accum.at[pl.ds(row0, _RPT), :])
        plsc.subcore_barrier()

        n_groups = -(-_MAXJ // _K)

        @pl.loop(0, n_groups)
        def group(gi):
            cids = [(gi * _K + b) * _NW + w for b in range(_K)]

            for b in range(_K):
                @pl.when(cids[b] < _NCHUNK)
                def _(b=b, cid=cids[b]):
                    base = cid * _CH
                    if not linear:
                        pltpu.async_copy(src_hbm.at[pl.ds(base, _CH)],
                                         sidx.at[b], isem)
                    pltpu.async_copy(dst_hbm.at[pl.ds(base, _CH)],
                                     didx.at[b], isem)

            for b in range(_K):
                @pl.when(cids[b] < _NCHUNK)
                def _(b=b, cid=cids[b]):
                    if not linear:
                        pltpu.make_async_copy(src_hbm.at[pl.ds(0, _CH)],
                                              sidx.at[b], isem).wait()
                    pltpu.make_async_copy(dst_hbm.at[pl.ds(0, _CH)],
                                          didx.at[b], isem).wait()
                    if linear:
                        base = cid * _CH
                        pltpu.async_copy(g_hbm.at[pl.ds(base, _CH), :],
                                         rows.at[b], gsem)
                    else:
                        pltpu.async_copy(g_hbm.at[sidx.at[b]], rows.at[b],
                                         gsem)

            for b in range(_K):
                @pl.when(cids[b] < _NCHUNK)
                def _(b=b, cid=cids[b]):
                    if linear:
                        pltpu.make_async_copy(g_hbm.at[pl.ds(0, _CH), :],
                                              rows.at[b], gsem).wait()
                    else:
                        pltpu.make_async_copy(g_hbm.at[sidx.at[b]],
                                              rows.at[b], gsem).wait()
                    pltpu.async_copy(rows.at[b], accum.at[didx.at[b]],
                                     ssem, add=True)

            for b in range(_K):
                @pl.when(cids[b] < _NCHUNK)
                def _(b=b):
                    pltpu.make_async_copy(rows.at[b], accum.at[didx.at[b]],
                                          ssem).wait()
        plsc.subcore_barrier()
        pltpu.sync_copy(accum.at[pl.ds(row0, _RPT), :],
                        out_hbm.at[c, pl.ds(row0, _RPT), :])

    return body


@functools.cache
def _sc_aggr_kernel(linear=False):
    return pl.kernel(
        _make_sc_body(linear),
        out_type=jax.ShapeDtypeStruct((_NC, _NPAD, _DF), jnp.float32),
        mesh=plsc.VectorSubcoreMesh(core_axis_name="c", subcore_axis_name="s"),
        scratch_types=[
            pltpu.VMEM((_K, _CH), jnp.int32),
            pltpu.VMEM((_K, _CH), jnp.int32),
            pltpu.VMEM((_K, _CH, _DF), jnp.float32),
            pltpu.VMEM_SHARED((_NPAD, _DF), jnp.float32),
            pltpu.SemaphoreType.DMA,
            pltpu.SemaphoreType.DMA,
            pltpu.SemaphoreType.DMA,
        ],
    )


_BN = 1000
_EB = 4000


def _relu_body(x_ref, gx_ref):
    gx_ref[...] = jnp.maximum(x_ref[...], 0.0)


def _relu_x(x):
    return pl.pallas_call(
        _relu_body,
        grid=(_N // _BN,),
        in_specs=[pl.BlockSpec((_BN, _DF), lambda i: (i, 0))],
        out_specs=pl.BlockSpec((_BN, _DF), lambda i: (i, 0)),
        out_shape=jax.ShapeDtypeStruct((_N, _DF), jnp.float32),
    )(x)


def _eapad_body(ea_ref, eap_ref):
    eap_ref[...] = jnp.concatenate(
        [jnp.maximum(ea_ref[...], 0.0),
         jnp.zeros((_EB, _DF - _DE), jnp.float32)], axis=1)


def _eapad(ea):
    return pl.pallas_call(
        _eapad_body,
        grid=(_E // _EB,),
        in_specs=[pl.BlockSpec((_EB, _DE), lambda i: (i, 0))],
        out_specs=pl.BlockSpec((_EB, _DF), lambda i: (i, 0)),
        out_shape=jax.ShapeDtypeStruct((_E, _DF), jnp.float32),
    )(ea)


def _layer_body(h_ref, p_ref, es_ref, W1a_ref, W1b_ref, b1_ref, W2_ref,
                b2_ref, o_ref):
    z = h_ref[...] + p_ref[0] + p_ref[1]
    es = es_ref[0] + es_ref[1]
    t = jnp.dot(z, W1a_ref[...], preferred_element_type=jnp.float32)
    t = t + jnp.dot(es, W1b_ref[...], preferred_element_type=jnp.float32)
    t = jnp.maximum(t + b1_ref[...], 0.0)
    o = jnp.dot(t, W2_ref[...], preferred_element_type=jnp.float32)
    o_ref[...] = jnp.maximum(o + b2_ref[...], 0.0)


def _layer(h, p, es, W1a, W1b, b1, W2, b2):
    return pl.pallas_call(
        _layer_body,
        grid=(_N // _BN,),
        in_specs=[
            pl.BlockSpec((_BN, _DF), lambda i: (i, 0)),
            pl.BlockSpec((_NC, _BN, _DF), lambda i: (0, i, 0)),
            pl.BlockSpec((_NC, _BN, _DF), lambda i: (0, i, 0)),
            pl.BlockSpec((_H, _H), lambda i: (0, 0)),
            pl.BlockSpec((_DF, _H), lambda i: (0, 0)),
            pl.BlockSpec((1, _H), lambda i: (0, 0)),
            pl.BlockSpec((_H, _H), lambda i: (0, 0)),
            pl.BlockSpec((1, _H), lambda i: (0, 0)),
        ],
        out_specs=pl.BlockSpec((_BN, _H), lambda i: (i, 0)),
        out_shape=jax.ShapeDtypeStruct((_N, _H), jnp.float32),
    )(h, p, es, W1a, W1b, b1, W2, b2)


def _pool_body(h_ref, b_ref, Wl_ref, bl_ref, o_ref, acc, cnt):
    i = pl.program_id(0)

    @pl.when(i == 0)
    def _():
        acc[...] = jnp.zeros_like(acc)
        cnt[...] = jnp.zeros_like(cnt)

    bvec = b_ref[0, 0]
    onehot = (bvec[:, None]
              == lax.broadcasted_iota(jnp.int32, (1, _G), 1)).astype(jnp.float32)
    acc[...] += lax.dot_general(onehot, h_ref[...], (((0,), (0,)), ((), ())),
                                preferred_element_type=jnp.float32)
    ones = jnp.ones((_BN, _H), jnp.float32)
    cnt[...] += lax.dot_general(onehot, ones, (((0,), (0,)), ((), ())),
                                preferred_element_type=jnp.float32)

    @pl.when(i == _N // _BN - 1)
    def _():
        pooled = acc[...] / jnp.maximum(cnt[...], 1.0)
        o_ref[...] = (jnp.dot(pooled, Wl_ref[...],
                              preferred_element_type=jnp.float32)
                      + bl_ref[...])


def _pool(h, batch3d, W_lin, b_lin2):
    return pl.pallas_call(
        _pool_body,
        grid=(_N // _BN,),
        in_specs=[
            pl.BlockSpec((_BN, _H), lambda i: (i, 0)),
            pl.BlockSpec((1, 1, _BN), lambda i: (i, 0, 0)),
            pl.BlockSpec((_H, _OUT), lambda i: (0, 0)),
            pl.BlockSpec((1, _OUT), lambda i: (0, 0)),
        ],
        out_specs=pl.BlockSpec((_G, _OUT), lambda i: (0, 0)),
        out_shape=jax.ShapeDtypeStruct((_G, _OUT), jnp.float32),
        scratch_shapes=[
            pltpu.VMEM((_G, _H), jnp.float32),
            pltpu.VMEM((_G, _H), jnp.float32),
        ],
    )(h, batch3d, W_lin, b_lin2)


def kernel(x, edge_index, edge_attr, batch,
           W1_0, b1_0, W2_0, b2_0,
           W1_1, b1_1, W2_1, b2_1,
           W1_2, b1_2, W2_2, b2_2,
           W_lin, b_lin):
    g0 = _relu_x(x)
    ea_pad = _eapad(edge_attr)

    z128 = jnp.zeros((_RPT, _DF), jnp.float32)
    src = edge_index[0]
    dst = edge_index[1]

    es = _sc_aggr_kernel(True)(ea_pad, dst, dst, z128)

    h = x
    g = g0
    for (W1, b1, W2, b2) in ((W1_0, b1_0, W2_0, b2_0),
                             (W1_1, b1_1, W2_1, b2_1),
                             (W1_2, b1_2, W2_2, b2_2)):
        p = _sc_aggr_kernel()(g, src, dst, z128)
        W1b_pad = jnp.concatenate(
            [W1[_H:], jnp.zeros((_DF - _DE, _H), jnp.float32)], axis=0)
        h = _layer(h, p, es, W1[:_H], W1b_pad, b1.reshape(1, _H),
                   W2, b2.reshape(1, _H))
        g = h

    batch3d = batch.reshape(_N // _BN, 1, _BN)
    return _pool(h, batch3d, W_lin, b_lin.reshape(1, _OUT))
